# baseline (speedup 1.0000x reference)
import functools

import jax
import jax.numpy as jnp
from jax.experimental import pallas as pl
from jax.experimental.pallas import tpu as pltpu

_BN_EPS = 1e-5


def _bn_act(y, aff_ref):
    m = jnp.mean(y, axis=0, keepdims=True)
    v = jnp.mean((y - m) ** 2, axis=0, keepdims=True)
    return jnp.maximum(
        (y - m) * (aff_ref[1:2, :] * jax.lax.rsqrt(v + _BN_EPS)) + aff_ref[2:3, :], 0.0)


def _se_kernel(w1t_ref, a1_ref, w2t_ref, a2_ref, x_hbm, o_ref,
               xs_ref, gate_ref, sems, *, nt_in, tin, tout, inv_hw):
    i = pl.program_id(0)

    @pl.when(i == 0)
    def _load_pool_excite():
        copies = [
            pltpu.make_async_copy(x_hbm.at[pl.ds(k * tin, tin)],
                                  xs_ref.at[pl.ds(k * tin, tin)],
                                  sems.at[k])
            for k in range(nt_in)
        ]
        for dma in copies:
            dma.start()
        s = None
        for k, dma in enumerate(copies):
            dma.wait()
            sk = jnp.sum(xs_ref[k * tin:(k + 1) * tin].astype(jnp.float32), axis=0)
            s = sk if s is None else s + sk
        a = s * inv_hw
        y1 = jax.lax.dot_general(a, w1t_ref[...], (((1,), (1,)), ((), ())),
                                 preferred_element_type=jnp.float32) + a1_ref[0:1, :]
        h1 = _bn_act(y1, a1_ref)
        y2 = jax.lax.dot_general(h1, w2t_ref[...], (((1,), (1,)), ((), ())),
                                 preferred_element_type=jnp.float32) + a2_ref[0:1, :]
        h2 = _bn_act(y2, a2_ref)
        gate_ref[...] = 1.0 / (1.0 + jnp.exp(-h2))

    g = gate_ref[...]
    o_ref[...] = (xs_ref[pl.ds(i * tout, tout)].astype(jnp.float32)
                  * g[None, :, :]).astype(o_ref.dtype)


def kernel(x, w1t, w2t, aff1, aff2):
    n, c, h, w = x.shape
    hw = h * w
    cr = w1t.shape[0]
    xt = x.transpose(2, 3, 0, 1).reshape(hw, n, c)

    tin = hw
    for cand in (196, 112, 98, 64, 56, 49, 28, 16, 8, 7, 4, 2, 1):
        if hw % cand == 0:
            tin = cand
            break
    nt_in = hw // tin
    tout = hw
    for cand in (98, 56, 49, 28, 16, 8, 7, 4, 2, 1):
        if hw % cand == 0:
            tout = cand
            break
    nt_out = hw // tout

    body = functools.partial(_se_kernel, nt_in=nt_in, tin=tin, tout=tout,
                             inv_hw=1.0 / float(hw))
    out = pl.pallas_call(
        body,
        out_shape=jax.ShapeDtypeStruct((hw, n, c), x.dtype),
        grid=(nt_out,),
        in_specs=[
            pl.BlockSpec((cr, c), lambda i: (0, 0)),
            pl.BlockSpec((3, cr), lambda i: (0, 0)),
            pl.BlockSpec((c, cr), lambda i: (0, 0)),
            pl.BlockSpec((3, c), lambda i: (0, 0)),
            pl.BlockSpec(memory_space=pl.ANY),
        ],
        out_specs=pl.BlockSpec((tout, n, c), lambda i: (i, 0, 0)),
        scratch_shapes=[
            pltpu.VMEM((hw, n, c), x.dtype),
            pltpu.VMEM((n, c), jnp.float32),
            pltpu.SemaphoreType.DMA((nt_in,)),
        ],
        compiler_params=pltpu.CompilerParams(
            dimension_semantics=("arbitrary",),
            vmem_limit_bytes=44 * 1024 * 1024),
        name="se_fused",
    )(w1t, aff1.T, w2t, aff2.T, xt)
    return out.reshape(h, w, n, c).transpose(2, 3, 0, 1)
